# Initial kernel scaffold; baseline (speedup 1.0000x reference)
import jax
import jax.numpy as jnp
from jax.experimental import pallas as pl


def kernel(beta, embed, slice_id, is_cp):
    raise NotImplementedError("write your pallas kernel here")


# speedup vs baseline: 6.5558x; 6.5558x over previous
import jax
import jax.numpy as jnp
from jax.experimental import pallas as pl
from jax.experimental.pallas import tpu as pltpu

AW = 1.0
RW = 1.5
TAU = 0.7
CLAMP = 20.0
K_INST = 64
CHUNK = 128

_HI = jax.lax.Precision.HIGHEST


def _clean(x):
    return jnp.where(jnp.isfinite(x), x, 0.0)


def _tc_body(beta_ref, et_ref, sid_ref, cp_ref, out_ref):
    b = pl.program_id(0)
    N = beta_ref.shape[1]

    bta = _clean(beta_ref[...])
    logits = jnp.clip(bta, -CLAMP, CLAMP) / TAU
    et = _clean(et_ref[...])
    sid = sid_ref[...]
    cpm = cp_ref[...] == 1

    inst = jax.lax.broadcasted_iota(jnp.int32, (K_INST, 1), 0)
    onehot = sid == inst
    cp_inst = onehot & cpm
    onef = onehot.astype(jnp.float32)
    cpif = cp_inst.astype(jnp.float32)

    cnt = jnp.sum(onef, axis=1, keepdims=True)
    cnt_cp = jnp.sum(cpif, axis=1, keepdims=True)
    cp_logit = jnp.sum(jnp.where(cp_inst, logits, 0.0), axis=1, keepdims=True)
    mx = jnp.max(jnp.where(onehot, logits, -1e30), axis=1, keepdims=True)
    mxs = jnp.maximum(mx, -100.0)
    denom = jnp.sum(jnp.where(onehot, jnp.exp(logits - mxs), 0.0),
                    axis=1, keepdims=True)

    niota = jax.lax.broadcasted_iota(jnp.int32, (K_INST, N), 1)
    fc = jnp.min(jnp.where(cp_inst, niota, N), axis=1, keepdims=True)
    ind = (niota == fc).astype(jnp.float32)
    cpv = jax.lax.dot_general(ind, et, (((1,), (1,)), ((), ())),
                              precision=_HI)
    cpn = jnp.sum(cpv * cpv, axis=1, keepdims=True)
    pn = jnp.sum(et * et, axis=0, keepdims=True)
    cpd = jax.lax.dot_general(cpv, et, (((1,), (0,)), ((), ())),
                              precision=_HI)
    d2 = cpn + pn - 2.0 * cpd
    mean_d = (jnp.sum(jnp.where(onehot, jnp.minimum(d2, 50.0), 0.0),
                      axis=1, keepdims=True)
              / jnp.maximum(cnt, 1.0))
    has_cp = cnt_cp > 0.0
    att_sum = jnp.sum(jnp.where(has_cp, mean_d, 0.0))
    att_cnt = jnp.sum(has_cp.astype(jnp.float32))
    attraction = jnp.where(att_cnt > 0.0,
                           AW * att_sum / jnp.maximum(att_cnt, 1.0), 0.0)

    sel = cnt_cp == 1.0
    p = jnp.exp(jnp.where(sel, cp_logit, 0.0) - jnp.where(sel, mxs, 0.0)) \
        / jnp.where(sel, denom, 1.0)
    ce = -jnp.log(p + 1e-9)
    nsel = jnp.sum(sel.astype(jnp.float32))
    beta_loss = jnp.sum(jnp.where(sel, ce, 0.0)) / jnp.maximum(nsel, 1.0)

    w = cpm.astype(jnp.float32)
    ones16 = jnp.ones((16, 1), jnp.float32)

    S = jnp.float32(0.0)
    for k in range(N // CHUNK):
        etc = et[:, k * CHUNK:(k + 1) * CHUNK]
        g = jax.lax.dot_general(etc, et, (((0,), (0,)), ((), ())),
                                precision=_HI)
        ncT = jax.lax.dot_general(etc * etc, ones16, (((0,), (0,)), ((), ())),
                                  precision=_HI)
        d2c = ncT + pn - 2.0 * g
        kmat = jnp.exp(-jnp.minimum(d2c, 50.0))
        t = jax.lax.dot_general(kmat, w, (((1,), (1,)), ((), ())),
                                precision=_HI)
        wc = w[:, k * CHUNK:(k + 1) * CHUNK]
        s = jax.lax.dot_general(wc, t, (((1,), (0,)), ((), ())),
                                precision=_HI)
        S = S + s[0, 0]
    ncp = jnp.sum(w)
    rep = jnp.where(ncp > 1.0, S / jnp.maximum(ncp * ncp, 1.0) * RW, 0.0)

    vf = (nsel > 0.0).astype(jnp.float32)
    loss_b = (beta_loss + attraction + rep) * vf

    lane = jax.lax.broadcasted_iota(jnp.int32, (1, 128), 1)
    row = (jnp.where(lane == 0, loss_b, 0.0)
           + jnp.where(lane == 1, beta_loss * vf, 0.0)
           + jnp.where(lane == 2, attraction * vf, 0.0)
           + jnp.where(lane == 3, rep * vf, 0.0)
           + jnp.where(lane == 4, vf, 0.0))

    @pl.when(b == 0)
    def _init():
        out_ref[...] = row

    @pl.when(b > 0)
    def _acc():
        out_ref[...] = out_ref[...] + row


def kernel(beta, embed, slice_id, is_cp):
    B, N, D = embed.shape
    beta2 = beta.reshape(B, 1, N)
    embed_t = jnp.swapaxes(embed, 1, 2)
    sid3 = slice_id.reshape(B, 1, N)
    cp3 = is_cp.reshape(B, 1, N)

    out = pl.pallas_call(
        _tc_body,
        grid=(B,),
        in_specs=[
            pl.BlockSpec((None, 1, N), lambda b: (b, 0, 0)),
            pl.BlockSpec((None, D, N), lambda b: (b, 0, 0)),
            pl.BlockSpec((None, 1, N), lambda b: (b, 0, 0)),
            pl.BlockSpec((None, 1, N), lambda b: (b, 0, 0)),
        ],
        out_specs=pl.BlockSpec((1, 128), lambda b: (0, 0)),
        out_shape=jax.ShapeDtypeStruct((1, 128), jnp.float32),
        compiler_params=pltpu.CompilerParams(
            dimension_semantics=("arbitrary",)),
    )(beta2, embed_t, sid3, cp3)

    total, bl, al, rl, vf = out[0, 0], out[0, 1], out[0, 2], out[0, 3], out[0, 4]
    safe = jnp.maximum(vf, 1.0)
    has = vf > 0.0
    z = jnp.float32(0.0)
    return (jnp.where(has, total / safe, z),
            jnp.where(has, bl / safe, z),
            jnp.where(has, al / safe, z),
            jnp.where(has, rl / safe, z))
